# baseline (speedup 1.0000x reference)
import functools

import jax
import jax.numpy as jnp
from jax import lax
from jax.experimental import pallas as pl
from jax.experimental.pallas import tpu as pltpu
from jax.experimental.pallas import tpu_sc as plsc

_B = 16384
_DIM = 32
_NC = 2
_NS = 16
_NW = _NC * _NS
_BPW = _B // _NW
_CHUNK = 128
_NCHUNK = _BPW // _CHUNK
_L = 16


def _sc_body(src_hbm, tgt_hbm, ns_hbm, ctx_hbm, out_hbm,
             sidx4, tidx4, soff, toff, srows, trows, outv, sem):
    wid = lax.axis_index("s") * _NC + lax.axis_index("c")
    base = wid * _BPW

    pltpu.sync_copy(src_hbm.at[pl.ds(base, _BPW)], soff)
    pltpu.sync_copy(tgt_hbm.at[pl.ds(base, _BPW)], toff)

    def split(i, carry):
        k = i * _L
        sv = soff[pl.ds(pl.multiple_of(k, _L), _L)]
        tv = toff[pl.ds(pl.multiple_of(k, _L), _L)]
        c = lax.div(i, jnp.int32(_CHUNK // _L))
        j = lax.rem(i, jnp.int32(_CHUNK // _L)) * _L
        sidx4[c, pl.ds(j, _L)] = lax.shift_right_logical(sv, 2)
        tidx4[c, pl.ds(j, _L)] = lax.shift_right_logical(tv, 2)
        soff[pl.ds(pl.multiple_of(k, _L), _L)] = (sv & 3) * _DIM
        toff[pl.ds(pl.multiple_of(k, _L), _L)] = (tv & 3) * _DIM
        return carry

    lax.fori_loop(0, _BPW // _L, split, 0)

    lane = lax.iota(jnp.int32, _L)

    def chunk(c, carry):
        cs = pltpu.async_copy(ns_hbm.at[sidx4.at[c]], srows, sem)
        ct = pltpu.async_copy(ctx_hbm.at[tidx4.at[c]], trows, sem)
        cs.wait()
        ct.wait()

        def group(g, carry2):
            i0 = c * _CHUNK + g * _L
            rows = g * _L + lane
            so = soff[pl.ds(i0, _L)]
            to = toff[pl.ds(i0, _L)]
            acc = jnp.zeros((_L,), jnp.float32)
            for d in range(_DIM):
                sv = plsc.load_gather(srows, [rows, so + d])
                tv = plsc.load_gather(trows, [rows, to + d])
                acc = acc + sv * tv
            outv[pl.ds(i0, _L)] = acc
            return carry2

        lax.fori_loop(0, _CHUNK // _L, group, 0)
        return carry

    lax.fori_loop(0, _NCHUNK, chunk, 0)

    pltpu.sync_copy(outv, out_hbm.at[pl.ds(base, _BPW)])


@functools.partial(
    pl.kernel,
    out_type=jax.ShapeDtypeStruct((_B,), jnp.float32),
    mesh=plsc.VectorSubcoreMesh(core_axis_name="c", subcore_axis_name="s"),
    scratch_types=[
        pltpu.VMEM((_NCHUNK, _CHUNK), jnp.int32),
        pltpu.VMEM((_NCHUNK, _CHUNK), jnp.int32),
        pltpu.VMEM((_BPW,), jnp.int32),
        pltpu.VMEM((_BPW,), jnp.int32),
        pltpu.VMEM((_CHUNK, 128), jnp.float32),
        pltpu.VMEM((_CHUNK, 128), jnp.float32),
        pltpu.VMEM((_BPW,), jnp.float32),
        pltpu.SemaphoreType.DMA,
    ],
    compiler_params=pltpu.CompilerParams(needs_layout_passes=False),
)
def _sc_dot(src_hbm, tgt_hbm, ns_hbm, ctx_hbm, out_hbm,
            sidx4, tidx4, soff, toff, srows, trows, outv, sem):
    _sc_body(src_hbm, tgt_hbm, ns_hbm, ctx_hbm, out_hbm,
             sidx4, tidx4, soff, toff, srows, trows, outv, sem)


def _loss_body(ip_ref, lab_ref, o_ref):
    x = lab_ref[...] * ip_ref[...]
    o_ref[0, 0] = -jnp.sum(jax.nn.log_sigmoid(x)) * (1.0 / _B)


_loss = pl.pallas_call(
    _loss_body,
    out_shape=jax.ShapeDtypeStruct((1, 1), jnp.float32),
    out_specs=pl.BlockSpec(memory_space=pltpu.MemorySpace.SMEM),
)


def kernel(source_node, target_node, label, nodes_embed, context_nodes_embed):
    ip = _sc_dot(source_node, target_node,
                 nodes_embed.reshape(250000, 128),
                 context_nodes_embed.reshape(250000, 128))
    loss = _loss(ip.reshape(128, 128), label.reshape(128, 128))
    return loss.reshape(())


# speedup vs baseline: 3.9608x; 3.9608x over previous
import functools

import jax
import jax.numpy as jnp
from jax import lax
from jax.experimental import pallas as pl
from jax.experimental.pallas import tpu as pltpu
from jax.experimental.pallas import tpu_sc as plsc

_B = 16384
_DIM = 32
_NC = 2
_NS = 16
_NW = _NC * _NS
_BPW = _B // _NW
_NB = 8
_L = 16


def _sc_body(src_hbm, tgt_hbm, ns_hbm, ctx_hbm, out_hbm,
             sidx, tidx, sbufs, tbufs, outv, sems, semt):
    wid = lax.axis_index("s") * _NC + lax.axis_index("c")
    base = wid * _BPW

    pltpu.sync_copy(src_hbm.at[pl.ds(base, _BPW)], sidx)
    pltpu.sync_copy(tgt_hbm.at[pl.ds(base, _BPW)], tidx)

    lane = lax.iota(jnp.int32, _L)

    def fire(rs, rt, b):
        cs = pl.multiple_of(lax.shift_right_logical(rs, 7) * 128, 128)
        ct = pl.multiple_of(lax.shift_right_logical(rt, 7) * 128, 128)
        pltpu.async_copy(ns_hbm.at[:, pl.ds(cs, 128)], sbufs[b], sems.at[b])
        pltpu.async_copy(ctx_hbm.at[:, pl.ds(ct, 128)], tbufs[b], semt.at[b])

    def drain(b):
        pltpu.make_async_copy(ns_hbm.at[:, pl.ds(0, 128)], sbufs[b], sems.at[b]).wait()
        pltpu.make_async_copy(ctx_hbm.at[:, pl.ds(0, 128)], tbufs[b], semt.at[b]).wait()

    def idx_vecs(g):
        off = pl.multiple_of(g * _L, _L)
        return sidx[pl.ds(off, _L)], tidx[pl.ds(off, _L)]

    siv0, tiv0 = idx_vecs(0)
    for b in range(_NB):
        fire(siv0[b], tiv0[b], b)

    def group(g, carry):
        siv, tiv = idx_vecs(g)
        snx, tnx = idx_vecs(jnp.minimum(g + 1, _BPW // _L - 1))
        acc = jnp.zeros((_L,), jnp.float32)
        for b in range(_L):
            slot = b % _NB
            drain(slot)
            sl = jnp.full((_L,), siv[b] & 127, jnp.int32)
            tl = jnp.full((_L,), tiv[b] & 127, jnp.int32)
            sv1 = plsc.load_gather(sbufs[slot], [lane, sl])
            sv2 = plsc.load_gather(sbufs[slot], [lane + _L, sl])
            tv1 = plsc.load_gather(tbufs[slot], [lane, tl])
            tv2 = plsc.load_gather(tbufs[slot], [lane + _L, tl])
            dot = jnp.sum(sv1 * tv1 + sv2 * tv2)
            acc = jnp.where(lane == b, dot, acc)

            if b < _NB:
                fire(siv[b + _NB], tiv[b + _NB], slot)
            else:
                rs, rt = snx[b - _NB], tnx[b - _NB]

                @pl.when(g < _BPW // _L - 1)
                def _():
                    fire(rs, rt, slot)

        outv[pl.ds(pl.multiple_of(g * _L, _L), _L)] = acc
        return carry

    lax.fori_loop(0, _BPW // _L, group, 0)

    pltpu.sync_copy(outv, out_hbm.at[pl.ds(base, _BPW)])


@functools.partial(
    pl.kernel,
    out_type=jax.ShapeDtypeStruct((_B,), jnp.float32),
    mesh=plsc.VectorSubcoreMesh(core_axis_name="c", subcore_axis_name="s"),
    scratch_types=[
        pltpu.VMEM((_BPW,), jnp.int32),
        pltpu.VMEM((_BPW,), jnp.int32),
        [pltpu.VMEM((_DIM, 128), jnp.float32)] * _NB,
        [pltpu.VMEM((_DIM, 128), jnp.float32)] * _NB,
        pltpu.VMEM((_BPW,), jnp.float32),
        pltpu.SemaphoreType.DMA((_NB,)),
        pltpu.SemaphoreType.DMA((_NB,)),
    ],
    compiler_params=pltpu.CompilerParams(needs_layout_passes=False),
)
def _sc_dot(src_hbm, tgt_hbm, ns_hbm, ctx_hbm, out_hbm,
            sidx, tidx, sbufs, tbufs, outv, sems, semt):
    _sc_body(src_hbm, tgt_hbm, ns_hbm, ctx_hbm, out_hbm,
             sidx, tidx, sbufs, tbufs, outv, sems, semt)


def _loss_body(ip_ref, lab_ref, o_ref):
    x = lab_ref[...] * ip_ref[...]
    o_ref[0, 0] = -jnp.sum(jax.nn.log_sigmoid(x)) * (1.0 / _B)


_loss = pl.pallas_call(
    _loss_body,
    out_shape=jax.ShapeDtypeStruct((1, 1), jnp.float32),
    out_specs=pl.BlockSpec(memory_space=pltpu.MemorySpace.SMEM),
)


def kernel(source_node, target_node, label, nodes_embed, context_nodes_embed):
    ip = _sc_dot(source_node, target_node,
                 nodes_embed.T, context_nodes_embed.T)
    loss = _loss(ip.reshape(128, 128), label.reshape(128, 128))
    return loss.reshape(())
